# Initial kernel scaffold; baseline (speedup 1.0000x reference)
import jax
import jax.numpy as jnp
from jax.experimental import pallas as pl


def kernel(x, edge_index, W, b):
    raise NotImplementedError("write your pallas kernel here")


# speedup vs baseline: 4.7389x; 4.7389x over previous
import functools

import jax
import jax.numpy as jnp
from jax import lax
from jax.experimental import pallas as pl
from jax.experimental.pallas import tpu as pltpu
from jax.experimental.pallas import tpu_sc as plsc

NC = 2
NS = 16
NW = NC * NS
CHUNK = 128


def _mesh():
    return plsc.VectorSubcoreMesh(core_axis_name="c", subcore_axis_name="s",
                                  num_cores=NC, num_subcores=NS)


def _make_deg_kernel(e_pad, n_pad, rpt):
    n_chunks = e_pad // (NW * CHUNK)
    ept = n_chunks * CHUNK

    @functools.partial(
        pl.kernel,
        out_type=jax.ShapeDtypeStruct((NC, n_pad, 16), jnp.float32),
        mesh=_mesh(),
        compiler_params=pltpu.CompilerParams(use_tc_tiling_on_sc=False),
        scratch_types=[
            pltpu.VMEM((CHUNK,), jnp.int32),
            pltpu.VMEM((CHUNK, 16), jnp.float32),
            pltpu.VMEM((rpt, 16), jnp.float32),
            pltpu.VMEM_SHARED((n_pad, 16), jnp.float32),
        ],
    )
    def deg_kernel(col_hbm, out_hbm, colbuf, ones_buf, zbuf, deg2d):
        c = lax.axis_index("c")
        s = lax.axis_index("s")
        wid = s * NC + c

        def fill_ones(i, _):
            ones_buf[i, :] = jnp.ones((16,), jnp.float32)
            return 0

        lax.fori_loop(0, CHUNK, fill_ones, 0)

        def fill_zero(i, _):
            zbuf[i, :] = jnp.zeros((16,), jnp.float32)
            return 0

        lax.fori_loop(0, rpt, fill_zero, 0)
        pltpu.sync_copy(zbuf, deg2d.at[pl.ds(s * rpt, rpt)])
        plsc.subcore_barrier()

        def ebody(k, _):
            base = wid * ept + k * CHUNK
            pltpu.sync_copy(col_hbm.at[pl.ds(base, CHUNK)], colbuf)
            pltpu.sync_copy(ones_buf, deg2d.at[colbuf], add=True)
            return 0

        lax.fori_loop(0, n_chunks, ebody, 0)
        plsc.subcore_barrier()
        pltpu.sync_copy(deg2d.at[pl.ds(s * rpt, rpt)], zbuf)
        pltpu.sync_copy(zbuf, out_hbm.at[c, pl.ds(s * rpt, rpt)])

    return deg_kernel


def _make_agg_kernel(e_pad, n_pad, rpt, ncls):
    n_chunks = e_pad // (NW * CHUNK)
    ept = n_chunks * CHUNK

    @functools.partial(
        pl.kernel,
        out_type=jax.ShapeDtypeStruct((NC, n_pad, ncls), jnp.float32),
        mesh=_mesh(),
        compiler_params=pltpu.CompilerParams(use_tc_tiling_on_sc=False),
        scratch_types=[
            pltpu.VMEM((CHUNK,), jnp.int32),
            pltpu.VMEM((CHUNK,), jnp.int32),
            pltpu.VMEM((CHUNK, ncls), jnp.float32),
            pltpu.VMEM((rpt, ncls), jnp.float32),
            pltpu.VMEM_SHARED((n_pad, ncls), jnp.float32),
            pltpu.SemaphoreType.DMA,
        ],
    )
    def agg_kernel(row_hbm, col_hbm, hd_hbm, out_hbm,
                   colbuf, rowbuf, rows_v, zbuf, agg, sem):
        c = lax.axis_index("c")
        s = lax.axis_index("s")
        wid = s * NC + c

        def fill_zero(i, _):
            for j in range(ncls // 16):
                zbuf[i, j * 16:(j + 1) * 16] = jnp.zeros((16,), jnp.float32)
            return 0

        lax.fori_loop(0, rpt, fill_zero, 0)
        pltpu.sync_copy(zbuf, agg.at[pl.ds(s * rpt, rpt)])
        plsc.subcore_barrier()

        def ebody(k, _):
            base = wid * ept + k * CHUNK
            pltpu.sync_copy(col_hbm.at[pl.ds(base, CHUNK)], colbuf)
            pltpu.sync_copy(row_hbm.at[pl.ds(base, CHUNK)], rowbuf)
            pltpu.async_copy(hd_hbm.at[colbuf], rows_v, sem).wait()
            pltpu.sync_copy(rows_v, agg.at[rowbuf], add=True)
            return 0

        lax.fori_loop(0, n_chunks, ebody, 0)
        plsc.subcore_barrier()
        pltpu.sync_copy(agg.at[pl.ds(s * rpt, rpt)], zbuf)
        pltpu.sync_copy(zbuf, out_hbm.at[c, pl.ds(s * rpt, rpt)])

    return agg_kernel


def _linear_body(n, bs, x_ref, w_ref, b_ref, degp_ref, hd_ref):
    i = pl.program_id(0)
    deg = degp_ref[0][:, 0:1] + degp_ref[1][:, 0:1] + 1.0
    dsq = lax.rsqrt(deg)
    h = lax.dot_general(x_ref[...], w_ref[...],
                        dimension_numbers=(((1,), (1,)), ((), ())),
                        preferred_element_type=jnp.float32) + b_ref[...]
    rid = i * bs + lax.broadcasted_iota(jnp.int32, (bs, 1), 0)
    hd_ref[...] = jnp.where(rid < n, dsq * h, 0.0)


def _finish_body(parts_ref, hd_ref, degp_ref, out_ref):
    deg = degp_ref[0][:, 0:1] + degp_ref[1][:, 0:1] + 1.0
    dsq = lax.rsqrt(deg)
    pre = dsq * (parts_ref[0] + parts_ref[1] + hd_ref[...])
    m = jnp.max(pre, axis=1, keepdims=True)
    e = jnp.exp(pre - m)
    ssum = jnp.sum(e, axis=1, keepdims=True)
    out_ref[...] = pre - m - jnp.log(ssum)


def kernel(x, edge_index, W, b):
    n, nfeat = x.shape
    ncls = W.shape[0]
    e = edge_index.shape[1]

    rpt = -(-(n + 1) // NS)
    rpt = -(-rpt // 32) * 32
    n_pad = rpt * NS
    e_pad = -(-e // (NW * CHUNK)) * (NW * CHUNK)

    row = edge_index[0]
    col = edge_index[1]
    pad_e = e_pad - e
    rowp = jnp.concatenate([row, jnp.full((pad_e,), n, jnp.int32)])
    colp = jnp.concatenate([col, jnp.full((pad_e,), n, jnp.int32)])
    x_pad = jnp.pad(x, ((0, n_pad - n), (0, 0)))
    b2 = b.reshape(1, ncls)

    degp = _make_deg_kernel(e_pad, n_pad, rpt)(colp)

    bs = 512
    grid = n_pad // bs
    hd = pl.pallas_call(
        functools.partial(_linear_body, n, bs),
        grid=(grid,),
        in_specs=[
            pl.BlockSpec((bs, nfeat), lambda i: (i, 0)),
            pl.BlockSpec((ncls, nfeat), lambda i: (0, 0)),
            pl.BlockSpec((1, ncls), lambda i: (0, 0)),
            pl.BlockSpec((NC, bs, 16), lambda i: (0, i, 0)),
        ],
        out_specs=pl.BlockSpec((bs, ncls), lambda i: (i, 0)),
        out_shape=jax.ShapeDtypeStruct((n_pad, ncls), jnp.float32),
    )(x_pad, W, b2, degp)

    parts = _make_agg_kernel(e_pad, n_pad, rpt, ncls)(rowp, colp, hd)

    out = pl.pallas_call(
        _finish_body,
        grid=(grid,),
        in_specs=[
            pl.BlockSpec((NC, bs, ncls), lambda i: (0, i, 0)),
            pl.BlockSpec((bs, ncls), lambda i: (i, 0)),
            pl.BlockSpec((NC, bs, 16), lambda i: (0, i, 0)),
        ],
        out_specs=pl.BlockSpec((bs, ncls), lambda i: (i, 0)),
        out_shape=jax.ShapeDtypeStruct((n_pad, ncls), jnp.float32),
    )(parts, hd, degp)

    return out[:n]


# speedup vs baseline: 5.3721x; 1.1336x over previous
import functools

import jax
import jax.numpy as jnp
from jax import lax
from jax.experimental import pallas as pl
from jax.experimental.pallas import tpu as pltpu
from jax.experimental.pallas import tpu_sc as plsc

NC = 2
NS = 16
NW = NC * NS
CHUNK = 128


def _mesh():
    return plsc.VectorSubcoreMesh(core_axis_name="c", subcore_axis_name="s",
                                  num_cores=NC, num_subcores=NS)


def _make_deg_kernel(e_pad, n_pad, rpt):
    nch = e_pad // (NW * CHUNK)

    @functools.partial(
        pl.kernel,
        out_type=jax.ShapeDtypeStruct((NC, n_pad, 16), jnp.float32),
        mesh=_mesh(),
        compiler_params=pltpu.CompilerParams(use_tc_tiling_on_sc=False),
        scratch_types=[
            pltpu.VMEM((nch, CHUNK), jnp.int32),
            pltpu.VMEM((CHUNK, 16), jnp.float32),
            pltpu.VMEM((rpt, 16), jnp.float32),
            pltpu.VMEM_SHARED((n_pad, 16), jnp.float32),
            pltpu.SemaphoreType.DMA,
            pltpu.SemaphoreType.DMA,
        ],
    )
    def deg_kernel(col_hbm, out_hbm, colbuf2, ones_buf, zbuf, deg2d, sg0, sg1):
        c = lax.axis_index("c")
        s = lax.axis_index("s")
        wid = s * NC + c

        def fill_ones(i, _):
            ones_buf[i, :] = jnp.ones((16,), jnp.float32)
            return 0

        lax.fori_loop(0, CHUNK, fill_ones, 0)

        def fill_zero(i, _):
            zbuf[i, :] = jnp.zeros((16,), jnp.float32)
            return 0

        lax.fori_loop(0, rpt, fill_zero, 0)
        pltpu.sync_copy(zbuf, deg2d.at[pl.ds(s * rpt, rpt)])
        pltpu.sync_copy(col_hbm.at[pl.ds(wid * nch, nch)], colbuf2)
        plsc.subcore_barrier()

        pltpu.async_copy(ones_buf, deg2d.at[colbuf2.at[0]], sg0, add=True)
        pltpu.async_copy(ones_buf, deg2d.at[colbuf2.at[1]], sg1, add=True)

        def ebody(k2, _):
            for b, sg in ((0, sg0), (1, sg1)):
                kk = k2 * 2 + b
                nxt = kk + 2

                @pl.when(nxt < nch)
                def _():
                    pltpu.async_copy(ones_buf, deg2d.at[colbuf2.at[nxt]],
                                     sg, add=True)

                pltpu.make_async_copy(
                    ones_buf, deg2d.at[pl.ds(0, CHUNK)], sg).wait()
            return 0

        lax.fori_loop(0, nch // 2, ebody, 0)
        plsc.subcore_barrier()
        pltpu.sync_copy(deg2d.at[pl.ds(s * rpt, rpt)], zbuf)
        pltpu.sync_copy(zbuf, out_hbm.at[c, pl.ds(s * rpt, rpt)])

    return deg_kernel


def _make_agg_kernel(e_pad, n_pad, rpt, ncls):
    nch = e_pad // (NW * CHUNK)

    @functools.partial(
        pl.kernel,
        out_type=jax.ShapeDtypeStruct((NC, n_pad, ncls), jnp.float32),
        mesh=_mesh(),
        compiler_params=pltpu.CompilerParams(use_tc_tiling_on_sc=False),
        scratch_types=[
            pltpu.VMEM((nch, CHUNK), jnp.int32),
            pltpu.VMEM((nch, CHUNK), jnp.int32),
            pltpu.VMEM((CHUNK, ncls), jnp.float32),
            pltpu.VMEM((CHUNK, ncls), jnp.float32),
            pltpu.VMEM((rpt, ncls), jnp.float32),
            pltpu.VMEM_SHARED((n_pad, ncls), jnp.float32),
            pltpu.SemaphoreType.DMA,
            pltpu.SemaphoreType.DMA,
        ],
    )
    def agg_kernel(row_hbm, col_hbm, hd_hbm, out_hbm,
                   colbuf2, rowbuf2, rows0, rows1, zbuf, agg, sg0, sg1):
        c = lax.axis_index("c")
        s = lax.axis_index("s")
        wid = s * NC + c

        def fill_zero(i, _):
            for j in range(ncls // 16):
                zbuf[i, j * 16:(j + 1) * 16] = jnp.zeros((16,), jnp.float32)
            return 0

        lax.fori_loop(0, rpt, fill_zero, 0)
        pltpu.sync_copy(zbuf, agg.at[pl.ds(s * rpt, rpt)])
        pltpu.sync_copy(col_hbm.at[pl.ds(wid * nch, nch)], colbuf2)
        pltpu.sync_copy(row_hbm.at[pl.ds(wid * nch, nch)], rowbuf2)
        plsc.subcore_barrier()

        pltpu.async_copy(hd_hbm.at[colbuf2.at[0]], rows0, sg0)
        pltpu.async_copy(hd_hbm.at[colbuf2.at[1]], rows1, sg1)

        def ebody(k2, _):
            for b, (rb, sg) in ((0, (rows0, sg0)), (1, (rows1, sg1))):
                kk = k2 * 2 + b
                pltpu.make_async_copy(
                    hd_hbm.at[pl.ds(0, CHUNK)], rb, sg).wait()
                pltpu.sync_copy(rb, agg.at[rowbuf2.at[kk]], add=True)
                nxt = kk + 2

                @pl.when(nxt < nch)
                def _():
                    pltpu.async_copy(hd_hbm.at[colbuf2.at[nxt]], rb, sg)
            return 0

        lax.fori_loop(0, nch // 2, ebody, 0)
        plsc.subcore_barrier()
        pltpu.sync_copy(agg.at[pl.ds(s * rpt, rpt)], zbuf)
        pltpu.sync_copy(zbuf, out_hbm.at[c, pl.ds(s * rpt, rpt)])

    return agg_kernel


def _linear_body(n, bs, x_ref, w_ref, b_ref, degp_ref, hd_ref):
    i = pl.program_id(0)
    deg = degp_ref[0][:, 0:1] + degp_ref[1][:, 0:1] + 1.0
    dsq = lax.rsqrt(deg)
    h = lax.dot_general(x_ref[...], w_ref[...],
                        dimension_numbers=(((1,), (1,)), ((), ())),
                        preferred_element_type=jnp.float32) + b_ref[...]
    rid = i * bs + lax.broadcasted_iota(jnp.int32, (bs, 1), 0)
    hd_ref[...] = jnp.where(rid < n, dsq * h, 0.0)


def _finish_body(parts_ref, hd_ref, degp_ref, out_ref):
    deg = degp_ref[0][:, 0:1] + degp_ref[1][:, 0:1] + 1.0
    dsq = lax.rsqrt(deg)
    pre = dsq * (parts_ref[0] + parts_ref[1] + hd_ref[...])
    m = jnp.max(pre, axis=1, keepdims=True)
    e = jnp.exp(pre - m)
    ssum = jnp.sum(e, axis=1, keepdims=True)
    out_ref[...] = pre - m - jnp.log(ssum)


def kernel(x, edge_index, W, b):
    n, nfeat = x.shape
    ncls = W.shape[0]
    e = edge_index.shape[1]

    rpt = -(-(n + 1) // NS)
    rpt = -(-rpt // 32) * 32
    n_pad = rpt * NS
    e_pad = -(-e // (2 * NW * CHUNK)) * (2 * NW * CHUNK)

    row = edge_index[0]
    col = edge_index[1]
    pad_e = e_pad - e
    rowp = jnp.concatenate(
        [row, jnp.full((pad_e,), n, jnp.int32)]).reshape(e_pad // CHUNK, CHUNK)
    colp = jnp.concatenate(
        [col, jnp.full((pad_e,), n, jnp.int32)]).reshape(e_pad // CHUNK, CHUNK)
    x_pad = jnp.pad(x, ((0, n_pad - n), (0, 0)))
    b2 = b.reshape(1, ncls)

    degp = _make_deg_kernel(e_pad, n_pad, rpt)(colp)

    bs = 512
    grid = n_pad // bs
    hd = pl.pallas_call(
        functools.partial(_linear_body, n, bs),
        grid=(grid,),
        in_specs=[
            pl.BlockSpec((bs, nfeat), lambda i: (i, 0)),
            pl.BlockSpec((ncls, nfeat), lambda i: (0, 0)),
            pl.BlockSpec((1, ncls), lambda i: (0, 0)),
            pl.BlockSpec((NC, bs, 16), lambda i: (0, i, 0)),
        ],
        out_specs=pl.BlockSpec((bs, ncls), lambda i: (i, 0)),
        out_shape=jax.ShapeDtypeStruct((n_pad, ncls), jnp.float32),
    )(x_pad, W, b2, degp)

    parts = _make_agg_kernel(e_pad, n_pad, rpt, ncls)(rowp, colp, hd)

    out = pl.pallas_call(
        _finish_body,
        grid=(grid,),
        in_specs=[
            pl.BlockSpec((NC, bs, ncls), lambda i: (0, i, 0)),
            pl.BlockSpec((bs, ncls), lambda i: (i, 0)),
            pl.BlockSpec((NC, bs, 16), lambda i: (0, i, 0)),
        ],
        out_specs=pl.BlockSpec((bs, ncls), lambda i: (i, 0)),
        out_shape=jax.ShapeDtypeStruct((n_pad, ncls), jnp.float32),
    )(parts, hd, degp)

    return out[:n]


# speedup vs baseline: 8.7397x; 1.6269x over previous
import functools

import jax
import jax.numpy as jnp
from jax import lax
from jax.experimental import pallas as pl
from jax.experimental.pallas import tpu as pltpu
from jax.experimental.pallas import tpu_sc as plsc

NC = 2
NS = 16
NW = NC * NS
CHUNK = 128


def _mesh():
    return plsc.VectorSubcoreMesh(core_axis_name="c", subcore_axis_name="s",
                                  num_cores=NC, num_subcores=NS)


def _make_deg_kernel(e_pad, n_pad, rpt):
    nch = e_pad // (NW * CHUNK)

    @functools.partial(
        pl.kernel,
        out_type=jax.ShapeDtypeStruct((NC, n_pad, 16), jnp.float32),
        mesh=_mesh(),
        compiler_params=pltpu.CompilerParams(use_tc_tiling_on_sc=False),
        scratch_types=[
            pltpu.VMEM((nch, CHUNK), jnp.int32),
            pltpu.VMEM((CHUNK, 16), jnp.float32),
            pltpu.VMEM((rpt, 16), jnp.float32),
            pltpu.VMEM_SHARED((n_pad, 16), jnp.float32),
            pltpu.SemaphoreType.DMA,
            pltpu.SemaphoreType.DMA,
        ],
    )
    def deg_kernel(col_hbm, out_hbm, colbuf2, ones_buf, zbuf, deg2d, sg0, sg1):
        c = lax.axis_index("c")
        s = lax.axis_index("s")
        wid = s * NC + c

        def fill_ones(i, _):
            ones_buf[i, :] = jnp.ones((16,), jnp.float32)
            return 0

        lax.fori_loop(0, CHUNK, fill_ones, 0)

        def fill_zero(i, _):
            zbuf[i, :] = jnp.zeros((16,), jnp.float32)
            return 0

        lax.fori_loop(0, rpt, fill_zero, 0)
        pltpu.sync_copy(zbuf, deg2d.at[pl.ds(s * rpt, rpt)])
        pltpu.sync_copy(col_hbm.at[pl.ds(wid * nch, nch)], colbuf2)
        plsc.subcore_barrier()

        pltpu.async_copy(ones_buf, deg2d.at[colbuf2.at[0]], sg0, add=True)
        pltpu.async_copy(ones_buf, deg2d.at[colbuf2.at[1]], sg1, add=True)

        def ebody(k2, _):
            for b, sg in ((0, sg0), (1, sg1)):
                kk = k2 * 2 + b
                nxt = kk + 2

                @pl.when(nxt < nch)
                def _():
                    pltpu.async_copy(ones_buf, deg2d.at[colbuf2.at[nxt]],
                                     sg, add=True)

                pltpu.make_async_copy(
                    ones_buf, deg2d.at[pl.ds(0, CHUNK)], sg).wait()
            return 0

        lax.fori_loop(0, nch // 2, ebody, 0)
        plsc.subcore_barrier()
        pltpu.sync_copy(deg2d.at[pl.ds(s * rpt, rpt)], zbuf)
        pltpu.sync_copy(zbuf, out_hbm.at[c, pl.ds(s * rpt, rpt)])

    return deg_kernel


def _make_agg_kernel(e_pad, n_pad, rpt, ncls):
    nch = e_pad // (NW * CHUNK)
    half = ncls // 2

    @functools.partial(
        pl.kernel,
        out_type=jax.ShapeDtypeStruct((2, NC, n_pad, half), jnp.float32),
        mesh=_mesh(),
        compiler_params=pltpu.CompilerParams(use_tc_tiling_on_sc=False),
        scratch_types=[
            pltpu.VMEM((nch, CHUNK), jnp.int32),
            pltpu.VMEM((nch, CHUNK), jnp.int32),
            pltpu.VMEM((CHUNK, half), jnp.float32),
            pltpu.VMEM((CHUNK, half), jnp.float32),
            pltpu.VMEM((rpt, half), jnp.float32),
            pltpu.VMEM_SHARED((n_pad, half), jnp.float32),
            pltpu.VMEM_SHARED((n_pad, half), jnp.float32),
            pltpu.SemaphoreType.DMA,
            pltpu.SemaphoreType.DMA,
        ],
    )
    def agg_kernel(row_hbm, col_hbm, hd_hbm, out_hbm,
                   colbuf2, rowbuf2, rows0, rows1, zbuf, agg, hds, sg0, sg1):
        c = lax.axis_index("c")
        s = lax.axis_index("s")
        wid = s * NC + c

        pltpu.sync_copy(col_hbm.at[pl.ds(wid * nch, nch)], colbuf2)
        pltpu.sync_copy(row_hbm.at[pl.ds(wid * nch, nch)], rowbuf2)

        for p in range(2):
            def fill_zero(i, _):
                for j in range(half // 16):
                    zbuf[i, j * 16:(j + 1) * 16] = jnp.zeros((16,),
                                                             jnp.float32)
                return 0

            lax.fori_loop(0, rpt, fill_zero, 0)
            pltpu.sync_copy(zbuf, agg.at[pl.ds(s * rpt, rpt)])
            pltpu.sync_copy(hd_hbm.at[p, pl.ds(s * rpt, rpt)], zbuf)
            pltpu.sync_copy(zbuf, hds.at[pl.ds(s * rpt, rpt)])
            plsc.subcore_barrier()

            pltpu.async_copy(hds.at[colbuf2.at[0]], rows0, sg0)
            pltpu.async_copy(hds.at[colbuf2.at[1]], rows1, sg1)

            def ebody(k2, _):
                for b, (rb, sg) in ((0, (rows0, sg0)), (1, (rows1, sg1))):
                    kk = k2 * 2 + b
                    pltpu.make_async_copy(
                        hd_hbm.at[p, pl.ds(0, CHUNK)], rb, sg).wait()
                    pltpu.sync_copy(rb, agg.at[rowbuf2.at[kk]], add=True)
                    nxt = kk + 2

                    @pl.when(nxt < nch)
                    def _():
                        pltpu.async_copy(hds.at[colbuf2.at[nxt]], rb, sg)
                return 0

            lax.fori_loop(0, nch // 2, ebody, 0)
            plsc.subcore_barrier()
            pltpu.sync_copy(agg.at[pl.ds(s * rpt, rpt)], zbuf)
            pltpu.sync_copy(zbuf, out_hbm.at[p, c, pl.ds(s * rpt, rpt)])
            plsc.subcore_barrier()

    return agg_kernel


def _linear_body(n, bs, ncls, x_ref, w_ref, b_ref, degp_ref, hd_ref):
    i = pl.program_id(0)
    deg = degp_ref[0][:, 0:1] + degp_ref[1][:, 0:1] + 1.0
    dsq = lax.rsqrt(deg)
    h = lax.dot_general(x_ref[...], w_ref[...],
                        dimension_numbers=(((1,), (1,)), ((), ())),
                        preferred_element_type=jnp.float32) + b_ref[...]
    rid = i * bs + lax.broadcasted_iota(jnp.int32, (bs, 1), 0)
    hd = jnp.where(rid < n, dsq * h, 0.0)
    half = ncls // 2
    hd_ref[0] = hd[:, :half]
    hd_ref[1] = hd[:, half:]


def _finish_body(parts_ref, hd_ref, degp_ref, out_ref):
    deg = degp_ref[0][:, 0:1] + degp_ref[1][:, 0:1] + 1.0
    dsq = lax.rsqrt(deg)
    pre0 = parts_ref[0, 0] + parts_ref[0, 1] + hd_ref[0]
    pre1 = parts_ref[1, 0] + parts_ref[1, 1] + hd_ref[1]
    pre = dsq * jnp.concatenate([pre0, pre1], axis=1)
    m = jnp.max(pre, axis=1, keepdims=True)
    e = jnp.exp(pre - m)
    ssum = jnp.sum(e, axis=1, keepdims=True)
    out_ref[...] = pre - m - jnp.log(ssum)


def kernel(x, edge_index, W, b):
    n, nfeat = x.shape
    ncls = W.shape[0]
    e = edge_index.shape[1]
    half = ncls // 2

    rpt = -(-(n + 1) // NS)
    rpt = -(-rpt // 32) * 32
    n_pad = rpt * NS
    e_pad = -(-e // (2 * NW * CHUNK)) * (2 * NW * CHUNK)

    row = edge_index[0]
    col = edge_index[1]
    pad_e = e_pad - e
    rowp = jnp.concatenate(
        [row, jnp.full((pad_e,), n, jnp.int32)]).reshape(e_pad // CHUNK, CHUNK)
    colp = jnp.concatenate(
        [col, jnp.full((pad_e,), n, jnp.int32)]).reshape(e_pad // CHUNK, CHUNK)
    x_pad = jnp.pad(x, ((0, n_pad - n), (0, 0)))
    b2 = b.reshape(1, ncls)

    degp = _make_deg_kernel(e_pad, n_pad, rpt)(colp)

    bs = 512
    grid = n_pad // bs
    hd = pl.pallas_call(
        functools.partial(_linear_body, n, bs, ncls),
        grid=(grid,),
        in_specs=[
            pl.BlockSpec((bs, nfeat), lambda i: (i, 0)),
            pl.BlockSpec((ncls, nfeat), lambda i: (0, 0)),
            pl.BlockSpec((1, ncls), lambda i: (0, 0)),
            pl.BlockSpec((NC, bs, 16), lambda i: (0, i, 0)),
        ],
        out_specs=pl.BlockSpec((2, bs, half), lambda i: (0, i, 0)),
        out_shape=jax.ShapeDtypeStruct((2, n_pad, half), jnp.float32),
    )(x_pad, W, b2, degp)

    parts = _make_agg_kernel(e_pad, n_pad, rpt, ncls)(rowp, colp, hd)

    out = pl.pallas_call(
        _finish_body,
        grid=(grid,),
        in_specs=[
            pl.BlockSpec((2, NC, bs, half), lambda i: (0, 0, i, 0)),
            pl.BlockSpec((2, bs, half), lambda i: (0, i, 0)),
            pl.BlockSpec((NC, bs, 16), lambda i: (0, i, 0)),
        ],
        out_specs=pl.BlockSpec((bs, ncls), lambda i: (i, 0)),
        out_shape=jax.ShapeDtypeStruct((n_pad, ncls), jnp.float32),
    )(parts, hd, degp)

    return out[:n]


# speedup vs baseline: 8.9151x; 1.0201x over previous
import functools

import jax
import jax.numpy as jnp
from jax import lax
from jax.experimental import pallas as pl
from jax.experimental.pallas import tpu as pltpu
from jax.experimental.pallas import tpu_sc as plsc

NC = 2
NS = 16
NW = NC * NS
CHUNK = 128


def _mesh():
    return plsc.VectorSubcoreMesh(core_axis_name="c", subcore_axis_name="s",
                                  num_cores=NC, num_subcores=NS)


def _make_deg_kernel(e_pad, n_pad, rpt):
    nch = e_pad // (NW * CHUNK)

    @functools.partial(
        pl.kernel,
        out_type=jax.ShapeDtypeStruct((NC, n_pad, 16), jnp.float32),
        mesh=_mesh(),
        compiler_params=pltpu.CompilerParams(use_tc_tiling_on_sc=False),
        scratch_types=[
            pltpu.VMEM((nch, CHUNK), jnp.int32),
            pltpu.VMEM((CHUNK, 16), jnp.float32),
            pltpu.VMEM((rpt, 16), jnp.float32),
            pltpu.VMEM_SHARED((n_pad, 16), jnp.float32),
            pltpu.SemaphoreType.DMA,
            pltpu.SemaphoreType.DMA,
        ],
    )
    def deg_kernel(col_hbm, out_hbm, colbuf2, ones_buf, zbuf, deg2d, sg0, sg1):
        c = lax.axis_index("c")
        s = lax.axis_index("s")
        wid = s * NC + c

        def fill_ones(i, _):
            ones_buf[i, :] = jnp.ones((16,), jnp.float32)
            return 0

        lax.fori_loop(0, CHUNK, fill_ones, 0)

        def fill_zero(i, _):
            zbuf[i, :] = jnp.zeros((16,), jnp.float32)
            return 0

        lax.fori_loop(0, rpt, fill_zero, 0)
        pltpu.sync_copy(zbuf, deg2d.at[pl.ds(s * rpt, rpt)])
        pltpu.sync_copy(col_hbm.at[pl.ds(wid * nch, nch)], colbuf2)
        plsc.subcore_barrier()

        pltpu.async_copy(ones_buf, deg2d.at[colbuf2.at[0]], sg0, add=True)
        pltpu.async_copy(ones_buf, deg2d.at[colbuf2.at[1]], sg1, add=True)

        def ebody(k2, _):
            for b, sg in ((0, sg0), (1, sg1)):
                kk = k2 * 2 + b
                nxt = kk + 2

                @pl.when(nxt < nch)
                def _():
                    pltpu.async_copy(ones_buf, deg2d.at[colbuf2.at[nxt]],
                                     sg, add=True)

                pltpu.make_async_copy(
                    ones_buf, deg2d.at[pl.ds(0, CHUNK)], sg).wait()
            return 0

        lax.fori_loop(0, nch // 2, ebody, 0)
        plsc.subcore_barrier()
        pltpu.sync_copy(deg2d.at[pl.ds(s * rpt, rpt)], zbuf)
        pltpu.sync_copy(zbuf, out_hbm.at[c, pl.ds(s * rpt, rpt)])

    return deg_kernel


def _make_agg_kernel(e_pad, n_pad, rpt, ncls):
    nch = e_pad // (NW * CHUNK)
    half = ncls // 2

    @functools.partial(
        pl.kernel,
        out_type=jax.ShapeDtypeStruct((2, NC, n_pad, half), jnp.float32),
        mesh=_mesh(),
        compiler_params=pltpu.CompilerParams(use_tc_tiling_on_sc=False),
        scratch_types=[
            pltpu.VMEM((nch, CHUNK), jnp.int32),
            pltpu.VMEM((nch, CHUNK), jnp.int32),
            pltpu.VMEM((CHUNK, half), jnp.float32),
            pltpu.VMEM((CHUNK, half), jnp.float32),
            pltpu.VMEM((CHUNK, half), jnp.float32),
            pltpu.VMEM((CHUNK, half), jnp.float32),
            pltpu.VMEM((rpt, half), jnp.float32),
            pltpu.VMEM_SHARED((n_pad, half), jnp.float32),
            pltpu.VMEM_SHARED((n_pad, half), jnp.float32),
            [pltpu.SemaphoreType.DMA] * 4,
            [pltpu.SemaphoreType.DMA] * 4,
        ],
    )
    def agg_kernel(row_hbm, col_hbm, hd_hbm, out_hbm,
                   colbuf2, rowbuf2, r0, r1, r2, r3, zbuf, agg, hds, gs, ss):
        c = lax.axis_index("c")
        s = lax.axis_index("s")
        wid = s * NC + c
        bufs = (r0, r1, r2, r3)

        pltpu.sync_copy(col_hbm.at[pl.ds(wid * nch, nch)], colbuf2)
        pltpu.sync_copy(row_hbm.at[pl.ds(wid * nch, nch)], rowbuf2)

        for p in range(2):
            def fill_zero(i, _):
                for j in range(half // 16):
                    zbuf[i, j * 16:(j + 1) * 16] = jnp.zeros((16,),
                                                             jnp.float32)
                return 0

            lax.fori_loop(0, rpt, fill_zero, 0)
            pltpu.sync_copy(zbuf, agg.at[pl.ds(s * rpt, rpt)])
            pltpu.sync_copy(hd_hbm.at[p, pl.ds(s * rpt, rpt)], zbuf)
            pltpu.sync_copy(zbuf, hds.at[pl.ds(s * rpt, rpt)])
            plsc.subcore_barrier()

            for b in range(4):
                pltpu.async_copy(hds.at[colbuf2.at[b]], bufs[b], gs[b])

            def ebody(k4, _):
                for b in range(4):
                    kk = k4 * 4 + b
                    rb = bufs[b]
                    pltpu.make_async_copy(
                        hd_hbm.at[p, pl.ds(0, CHUNK)], rb, gs[b]).wait()
                    pltpu.async_copy(rb, agg.at[rowbuf2.at[kk]], ss[b],
                                     add=True)
                    nxt = kk + 4

                    @pl.when(nxt < nch)
                    def _():
                        pltpu.make_async_copy(
                            rb, agg.at[pl.ds(0, CHUNK)], ss[b]).wait()
                        pltpu.async_copy(hds.at[colbuf2.at[nxt]], rb, gs[b])
                return 0

            lax.fori_loop(0, nch // 4, ebody, 0)
            for b in range(4):
                pltpu.make_async_copy(
                    bufs[b], agg.at[pl.ds(0, CHUNK)], ss[b]).wait()
            plsc.subcore_barrier()
            pltpu.sync_copy(agg.at[pl.ds(s * rpt, rpt)], zbuf)
            pltpu.sync_copy(zbuf, out_hbm.at[p, c, pl.ds(s * rpt, rpt)])
            plsc.subcore_barrier()

    return agg_kernel


def _linear_body(n, bs, ncls, x_ref, w_ref, b_ref, degp_ref, hd_ref):
    i = pl.program_id(0)
    deg = degp_ref[0][:, 0:1] + degp_ref[1][:, 0:1] + 1.0
    dsq = lax.rsqrt(deg)
    h = lax.dot_general(x_ref[...], w_ref[...],
                        dimension_numbers=(((1,), (1,)), ((), ())),
                        preferred_element_type=jnp.float32) + b_ref[...]
    rid = i * bs + lax.broadcasted_iota(jnp.int32, (bs, 1), 0)
    hd = jnp.where(rid < n, dsq * h, 0.0)
    half = ncls // 2
    hd_ref[0] = hd[:, :half]
    hd_ref[1] = hd[:, half:]


def _finish_body(parts_ref, hd_ref, degp_ref, out_ref):
    deg = degp_ref[0][:, 0:1] + degp_ref[1][:, 0:1] + 1.0
    dsq = lax.rsqrt(deg)
    pre0 = parts_ref[0, 0] + parts_ref[0, 1] + hd_ref[0]
    pre1 = parts_ref[1, 0] + parts_ref[1, 1] + hd_ref[1]
    pre = dsq * jnp.concatenate([pre0, pre1], axis=1)
    m = jnp.max(pre, axis=1, keepdims=True)
    e = jnp.exp(pre - m)
    ssum = jnp.sum(e, axis=1, keepdims=True)
    out_ref[...] = pre - m - jnp.log(ssum)


def kernel(x, edge_index, W, b):
    n, nfeat = x.shape
    ncls = W.shape[0]
    e = edge_index.shape[1]
    half = ncls // 2

    rpt = -(-(n + 1) // NS)
    rpt = -(-rpt // 32) * 32
    n_pad = rpt * NS
    e_pad = -(-e // (4 * NW * CHUNK)) * (4 * NW * CHUNK)

    row = edge_index[0]
    col = edge_index[1]
    pad_e = e_pad - e
    rowp = jnp.concatenate(
        [row, jnp.full((pad_e,), n, jnp.int32)]).reshape(e_pad // CHUNK, CHUNK)
    colp = jnp.concatenate(
        [col, jnp.full((pad_e,), n, jnp.int32)]).reshape(e_pad // CHUNK, CHUNK)
    x_pad = jnp.pad(x, ((0, n_pad - n), (0, 0)))
    b2 = b.reshape(1, ncls)

    degp = _make_deg_kernel(e_pad, n_pad, rpt)(colp)

    bs = 512
    grid = n_pad // bs
    hd = pl.pallas_call(
        functools.partial(_linear_body, n, bs, ncls),
        grid=(grid,),
        in_specs=[
            pl.BlockSpec((bs, nfeat), lambda i: (i, 0)),
            pl.BlockSpec((ncls, nfeat), lambda i: (0, 0)),
            pl.BlockSpec((1, ncls), lambda i: (0, 0)),
            pl.BlockSpec((NC, bs, 16), lambda i: (0, i, 0)),
        ],
        out_specs=pl.BlockSpec((2, bs, half), lambda i: (0, i, 0)),
        out_shape=jax.ShapeDtypeStruct((2, n_pad, half), jnp.float32),
    )(x_pad, W, b2, degp)

    parts = _make_agg_kernel(e_pad, n_pad, rpt, ncls)(rowp, colp, hd)

    out = pl.pallas_call(
        _finish_body,
        grid=(grid,),
        in_specs=[
            pl.BlockSpec((2, NC, bs, half), lambda i: (0, 0, i, 0)),
            pl.BlockSpec((2, bs, half), lambda i: (0, i, 0)),
            pl.BlockSpec((NC, bs, 16), lambda i: (0, i, 0)),
        ],
        out_specs=pl.BlockSpec((bs, ncls), lambda i: (i, 0)),
        out_shape=jax.ShapeDtypeStruct((n_pad, ncls), jnp.float32),
    )(parts, hd, degp)

    return out[:n]


# speedup vs baseline: 9.4360x; 1.0584x over previous
import functools

import jax
import jax.numpy as jnp
from jax import lax
from jax.experimental import pallas as pl
from jax.experimental.pallas import tpu as pltpu
from jax.experimental.pallas import tpu_sc as plsc

NC = 2
NS = 16
NW = NC * NS
CHUNK = 128


def _mesh():
    return plsc.VectorSubcoreMesh(core_axis_name="c", subcore_axis_name="s",
                                  num_cores=NC, num_subcores=NS)


def _make_deg_kernel(e_pad, n_pad, rpt):
    nch = e_pad // (NW * CHUNK)
    nrows = n_pad // CHUNK
    rpc = nrows // NS

    @functools.partial(
        pl.kernel,
        out_type=jax.ShapeDtypeStruct((NC, nrows, CHUNK), jnp.float32),
        mesh=_mesh(),
        compiler_params=pltpu.CompilerParams(use_tc_tiling_on_sc=False,
                                             needs_layout_passes=False),
        scratch_types=[
            pltpu.VMEM((nch, CHUNK), jnp.int32),
            pltpu.VMEM((n_pad,), jnp.float32),
            pltpu.VMEM((nrows, CHUNK), jnp.float32),
            pltpu.VMEM((nrows,), jnp.int32),
            pltpu.VMEM_SHARED((nrows, CHUNK), jnp.float32),
        ],
    )
    def deg_kernel(col_hbm, out_hbm, colbuf2, degflat, degloc, ibuf, deg2):
        c = lax.axis_index("c")
        s = lax.axis_index("s")
        wid = s * NC + c

        def fill_zero(i, _):
            for j in range(CHUNK // 16):
                degflat[pl.ds(i * CHUNK + j * 16, 16)] = jnp.zeros(
                    (16,), jnp.float32)
                degloc[i, j * 16:(j + 1) * 16] = jnp.zeros((16,), jnp.float32)
            return 0

        lax.fori_loop(0, nrows, fill_zero, 0)
        for i in range(nrows // 16):
            ibuf[i * 16:(i + 1) * 16] = lax.iota(jnp.int32, 16) + i * 16
        pltpu.sync_copy(degloc.at[pl.ds(0, rpc)], deg2.at[pl.ds(s * rpc, rpc)])
        pltpu.sync_copy(col_hbm.at[pl.ds(wid * nch, nch)], colbuf2)
        plsc.subcore_barrier()

        ones16 = jnp.ones((16,), jnp.float32)

        def ebody(k, _):
            for j in range(CHUNK // 16):
                idx = colbuf2[k, j * 16:(j + 1) * 16]
                plsc.addupdate_scatter(degflat, [idx], ones16)
            return 0

        lax.fori_loop(0, nch, ebody, 0)

        def repack(i, _):
            for j in range(CHUNK // 16):
                degloc[i, j * 16:(j + 1) * 16] = degflat[
                    pl.ds(i * CHUNK + j * 16, 16)]
            return 0

        lax.fori_loop(0, nrows, repack, 0)
        pltpu.sync_copy(degloc, deg2.at[ibuf], add=True)
        plsc.subcore_barrier()
        pltpu.sync_copy(deg2.at[pl.ds(s * rpc, rpc)],
                        degloc.at[pl.ds(0, rpc)])
        pltpu.sync_copy(degloc.at[pl.ds(0, rpc)],
                        out_hbm.at[c, pl.ds(s * rpc, rpc)])

    return deg_kernel


def _make_agg_kernel(e_pad, n_pad, rpt, ncls):
    nch = e_pad // (NW * CHUNK)
    half = ncls // 2

    @functools.partial(
        pl.kernel,
        out_type=jax.ShapeDtypeStruct((2, NC, n_pad, half), jnp.float32),
        mesh=_mesh(),
        compiler_params=pltpu.CompilerParams(use_tc_tiling_on_sc=False),
        scratch_types=[
            pltpu.VMEM((nch, CHUNK), jnp.int32),
            pltpu.VMEM((nch, CHUNK), jnp.int32),
            pltpu.VMEM((CHUNK, half), jnp.float32),
            pltpu.VMEM((CHUNK, half), jnp.float32),
            pltpu.VMEM((CHUNK, half), jnp.float32),
            pltpu.VMEM((CHUNK, half), jnp.float32),
            pltpu.VMEM((rpt, half), jnp.float32),
            pltpu.VMEM_SHARED((n_pad, half), jnp.float32),
            pltpu.VMEM_SHARED((n_pad, half), jnp.float32),
            [pltpu.SemaphoreType.DMA] * 4,
            [pltpu.SemaphoreType.DMA] * 4,
        ],
    )
    def agg_kernel(row_hbm, col_hbm, hd_hbm, out_hbm,
                   colbuf2, rowbuf2, r0, r1, r2, r3, zbuf, agg, hds, gs, ss):
        c = lax.axis_index("c")
        s = lax.axis_index("s")
        wid = s * NC + c
        bufs = (r0, r1, r2, r3)

        pltpu.sync_copy(col_hbm.at[pl.ds(wid * nch, nch)], colbuf2)
        pltpu.sync_copy(row_hbm.at[pl.ds(wid * nch, nch)], rowbuf2)

        for p in range(2):
            def fill_zero(i, _):
                for j in range(half // 16):
                    zbuf[i, j * 16:(j + 1) * 16] = jnp.zeros((16,),
                                                             jnp.float32)
                return 0

            lax.fori_loop(0, rpt, fill_zero, 0)
            pltpu.sync_copy(zbuf, agg.at[pl.ds(s * rpt, rpt)])
            pltpu.sync_copy(hd_hbm.at[p, pl.ds(s * rpt, rpt)], zbuf)
            pltpu.sync_copy(zbuf, hds.at[pl.ds(s * rpt, rpt)])
            plsc.subcore_barrier()

            for b in range(4):
                pltpu.async_copy(hds.at[colbuf2.at[b]], bufs[b], gs[b])

            def ebody(k4, _):
                for b in range(4):
                    kk = k4 * 4 + b
                    rb = bufs[b]
                    pltpu.make_async_copy(
                        hd_hbm.at[p, pl.ds(0, CHUNK)], rb, gs[b]).wait()
                    pltpu.async_copy(rb, agg.at[rowbuf2.at[kk]], ss[b],
                                     add=True)
                    nxt = kk + 4

                    @pl.when(nxt < nch)
                    def _():
                        pltpu.make_async_copy(
                            rb, agg.at[pl.ds(0, CHUNK)], ss[b]).wait()
                        pltpu.async_copy(hds.at[colbuf2.at[nxt]], rb, gs[b])
                return 0

            lax.fori_loop(0, nch // 4, ebody, 0)
            for b in range(4):
                pltpu.make_async_copy(
                    bufs[b], agg.at[pl.ds(0, CHUNK)], ss[b]).wait()
            plsc.subcore_barrier()
            pltpu.sync_copy(agg.at[pl.ds(s * rpt, rpt)], zbuf)
            pltpu.sync_copy(zbuf, out_hbm.at[p, c, pl.ds(s * rpt, rpt)])
            plsc.subcore_barrier()

    return agg_kernel


def _linear_body(n, bs, ncls, x_ref, w_ref, b_ref, degp_ref, hd_ref):
    i = pl.program_id(0)
    deg = degp_ref[0][:, 0:1] + degp_ref[1][:, 0:1] + 1.0
    dsq = lax.rsqrt(deg)
    h = lax.dot_general(x_ref[...], w_ref[...],
                        dimension_numbers=(((1,), (1,)), ((), ())),
                        preferred_element_type=jnp.float32) + b_ref[...]
    rid = i * bs + lax.broadcasted_iota(jnp.int32, (bs, 1), 0)
    hd = jnp.where(rid < n, dsq * h, 0.0)
    half = ncls // 2
    hd_ref[0] = hd[:, :half]
    hd_ref[1] = hd[:, half:]


def _finish_body(parts_ref, hd_ref, degp_ref, out_ref):
    deg = degp_ref[0][:, 0:1] + degp_ref[1][:, 0:1] + 1.0
    dsq = lax.rsqrt(deg)
    pre0 = parts_ref[0, 0] + parts_ref[0, 1] + hd_ref[0]
    pre1 = parts_ref[1, 0] + parts_ref[1, 1] + hd_ref[1]
    pre = dsq * jnp.concatenate([pre0, pre1], axis=1)
    m = jnp.max(pre, axis=1, keepdims=True)
    e = jnp.exp(pre - m)
    ssum = jnp.sum(e, axis=1, keepdims=True)
    out_ref[...] = pre - m - jnp.log(ssum)


def kernel(x, edge_index, W, b):
    n, nfeat = x.shape
    ncls = W.shape[0]
    e = edge_index.shape[1]
    half = ncls // 2

    rpt = -(-(n + 1) // NS)
    rpt = -(-rpt // 32) * 32
    n_pad = rpt * NS
    e_pad = -(-e // (4 * NW * CHUNK)) * (4 * NW * CHUNK)

    row = edge_index[0]
    col = edge_index[1]
    pad_e = e_pad - e
    rowp = jnp.concatenate(
        [row, jnp.full((pad_e,), n, jnp.int32)]).reshape(e_pad // CHUNK, CHUNK)
    colp = jnp.concatenate(
        [col, jnp.full((pad_e,), n, jnp.int32)]).reshape(e_pad // CHUNK, CHUNK)
    x_pad = jnp.pad(x, ((0, n_pad - n), (0, 0)))
    b2 = b.reshape(1, ncls)

    degp = _make_deg_kernel(e_pad, n_pad, rpt)(colp).reshape(NC, n_pad, 1)

    bs = 1024
    grid = n_pad // bs
    hd = pl.pallas_call(
        functools.partial(_linear_body, n, bs, ncls),
        grid=(grid,),
        in_specs=[
            pl.BlockSpec((bs, nfeat), lambda i: (i, 0)),
            pl.BlockSpec((ncls, nfeat), lambda i: (0, 0)),
            pl.BlockSpec((1, ncls), lambda i: (0, 0)),
            pl.BlockSpec((NC, bs, 1), lambda i: (0, i, 0)),
        ],
        out_specs=pl.BlockSpec((2, bs, half), lambda i: (0, i, 0)),
        out_shape=jax.ShapeDtypeStruct((2, n_pad, half), jnp.float32),
    )(x_pad, W, b2, degp)

    parts = _make_agg_kernel(e_pad, n_pad, rpt, ncls)(rowp, colp, hd)

    out = pl.pallas_call(
        _finish_body,
        grid=(grid,),
        in_specs=[
            pl.BlockSpec((2, NC, bs, half), lambda i: (0, 0, i, 0)),
            pl.BlockSpec((2, bs, half), lambda i: (0, i, 0)),
            pl.BlockSpec((NC, bs, 1), lambda i: (0, i, 0)),
        ],
        out_specs=pl.BlockSpec((bs, ncls), lambda i: (i, 0)),
        out_shape=jax.ShapeDtypeStruct((n_pad, ncls), jnp.float32),
    )(parts, hd, degp)

    return out[:n]


# speedup vs baseline: 10.1702x; 1.0778x over previous
import functools

import jax
import jax.numpy as jnp
from jax import lax
from jax.experimental import pallas as pl
from jax.experimental.pallas import tpu as pltpu
from jax.experimental.pallas import tpu_sc as plsc

NC = 2
NS = 16
NW = NC * NS
CHUNK = 128


def _mesh():
    return plsc.VectorSubcoreMesh(core_axis_name="c", subcore_axis_name="s",
                                  num_cores=NC, num_subcores=NS)


def _make_deg_kernel(e_pad, n_pad, rpt):
    nch = e_pad // (NW * CHUNK)
    nrows = n_pad // CHUNK
    rpc = nrows // NS

    @functools.partial(
        pl.kernel,
        out_type=jax.ShapeDtypeStruct((NC, nrows, CHUNK), jnp.float32),
        mesh=_mesh(),
        compiler_params=pltpu.CompilerParams(use_tc_tiling_on_sc=False,
                                             needs_layout_passes=False),
        scratch_types=[
            pltpu.VMEM((nch, CHUNK), jnp.int32),
            pltpu.VMEM((n_pad,), jnp.float32),
            pltpu.VMEM((nrows, CHUNK), jnp.float32),
            pltpu.VMEM((nrows,), jnp.int32),
            pltpu.VMEM_SHARED((nrows, CHUNK), jnp.float32),
        ],
    )
    def deg_kernel(col_hbm, out_hbm, colbuf2, degflat, degloc, ibuf, deg2):
        c = lax.axis_index("c")
        s = lax.axis_index("s")
        wid = s * NC + c

        def fill_zero(i, _):
            for j in range(CHUNK // 16):
                degflat[pl.ds(i * CHUNK + j * 16, 16)] = jnp.zeros(
                    (16,), jnp.float32)
                degloc[i, j * 16:(j + 1) * 16] = jnp.zeros((16,), jnp.float32)
            return 0

        lax.fori_loop(0, nrows, fill_zero, 0)
        for i in range(nrows // 16):
            ibuf[i * 16:(i + 1) * 16] = lax.iota(jnp.int32, 16) + i * 16
        pltpu.sync_copy(degloc.at[pl.ds(0, rpc)], deg2.at[pl.ds(s * rpc, rpc)])
        pltpu.sync_copy(col_hbm.at[pl.ds(wid * nch, nch)], colbuf2)
        plsc.subcore_barrier()

        ones16 = jnp.ones((16,), jnp.float32)

        def ebody(k, _):
            for j in range(CHUNK // 16):
                idx = colbuf2[k, j * 16:(j + 1) * 16]
                plsc.addupdate_scatter(degflat, [idx], ones16)
            return 0

        lax.fori_loop(0, nch, ebody, 0)

        def repack(i, _):
            for j in range(CHUNK // 16):
                degloc[i, j * 16:(j + 1) * 16] = degflat[
                    pl.ds(i * CHUNK + j * 16, 16)]
            return 0

        lax.fori_loop(0, nrows, repack, 0)
        pltpu.sync_copy(degloc, deg2.at[ibuf], add=True)
        plsc.subcore_barrier()
        pltpu.sync_copy(deg2.at[pl.ds(s * rpc, rpc)],
                        degloc.at[pl.ds(0, rpc)])
        pltpu.sync_copy(degloc.at[pl.ds(0, rpc)],
                        out_hbm.at[c, pl.ds(s * rpc, rpc)])

    return deg_kernel


def _make_agg_kernel(e_pad, n_pad, rpt, ncls):
    nch = e_pad // (NW * CHUNK)
    half = ncls // 2

    @functools.partial(
        pl.kernel,
        out_type=jax.ShapeDtypeStruct((2, NC, n_pad, half), jnp.float32),
        mesh=_mesh(),
        compiler_params=pltpu.CompilerParams(use_tc_tiling_on_sc=False),
        scratch_types=[
            pltpu.VMEM((nch, CHUNK), jnp.int32),
            pltpu.VMEM((nch, CHUNK), jnp.int32),
            pltpu.VMEM((CHUNK, half), jnp.float32),
            pltpu.VMEM((CHUNK, half), jnp.float32),
            pltpu.VMEM((CHUNK, half), jnp.float32),
            pltpu.VMEM((CHUNK, half), jnp.float32),
            pltpu.VMEM((rpt, half), jnp.float32),
            pltpu.VMEM_SHARED((n_pad, half), jnp.float32),
            pltpu.VMEM_SHARED((n_pad, half), jnp.float32),
            [pltpu.SemaphoreType.DMA] * 4,
            [pltpu.SemaphoreType.DMA] * 4,
        ],
    )
    def agg_kernel(row_hbm, col_hbm, hd_hbm, out_hbm,
                   colbuf2, rowbuf2, r0, r1, r2, r3, zbuf, agg, hds, gs, ss):
        c = lax.axis_index("c")
        s = lax.axis_index("s")
        wid = s * NC + c
        bufs = (r0, r1, r2, r3)

        pltpu.sync_copy(col_hbm.at[pl.ds(wid * nch, nch)], colbuf2)
        pltpu.sync_copy(row_hbm.at[pl.ds(wid * nch, nch)], rowbuf2)

        for p in range(2):
            def fill_zero(i, _):
                for j in range(half // 16):
                    zbuf[i, j * 16:(j + 1) * 16] = jnp.zeros((16,),
                                                             jnp.float32)
                return 0

            @pl.when(c == 1)
            def _():
                lax.fori_loop(0, rpt, fill_zero, 0)
                pltpu.sync_copy(zbuf, agg.at[pl.ds(s * rpt, rpt)])

            pltpu.sync_copy(hd_hbm.at[p, pl.ds(s * rpt, rpt)], zbuf)
            pltpu.sync_copy(zbuf, hds.at[pl.ds(s * rpt, rpt)])

            @pl.when(c == 0)
            def _():
                pltpu.sync_copy(zbuf, agg.at[pl.ds(s * rpt, rpt)])

            plsc.subcore_barrier()

            for b in range(4):
                pltpu.async_copy(hds.at[colbuf2.at[b]], bufs[b], gs[b])

            def ebody(k4, _):
                for b in range(4):
                    kk = k4 * 4 + b
                    rb = bufs[b]
                    pltpu.make_async_copy(
                        hd_hbm.at[p, pl.ds(0, CHUNK)], rb, gs[b]).wait()
                    pltpu.async_copy(rb, agg.at[rowbuf2.at[kk]], ss[b],
                                     add=True)
                    nxt = kk + 4

                    @pl.when(nxt < nch)
                    def _():
                        pltpu.make_async_copy(
                            rb, agg.at[pl.ds(0, CHUNK)], ss[b]).wait()
                        pltpu.async_copy(hds.at[colbuf2.at[nxt]], rb, gs[b])
                return 0

            lax.fori_loop(0, nch // 4, ebody, 0)
            for b in range(4):
                pltpu.make_async_copy(
                    bufs[b], agg.at[pl.ds(0, CHUNK)], ss[b]).wait()
            plsc.subcore_barrier()
            pltpu.sync_copy(agg.at[pl.ds(s * rpt, rpt)], zbuf)
            pltpu.sync_copy(zbuf, out_hbm.at[p, c, pl.ds(s * rpt, rpt)])
            plsc.subcore_barrier()

    return agg_kernel


def _unpack_deg(dp, bs):
    bsp = dp.shape[0]
    rid = lax.broadcasted_iota(jnp.int32, (bs, 1), 0)
    rsel = (lax.broadcasted_iota(jnp.int32, (bs, bsp), 1)
            == (rid >> 7)).astype(jnp.float32)
    d = lax.dot_general(rsel, dp, dimension_numbers=(((1,), (0,)), ((), ())),
                        preferred_element_type=jnp.float32)
    lsel = (lax.broadcasted_iota(jnp.int32, (bs, CHUNK), 1) == (rid & 127))
    return jnp.sum(jnp.where(lsel, d, 0.0), axis=1, keepdims=True)


def _linear_body(n, bs, ncls, x_ref, w_ref, b_ref, degp_ref, hd_ref):
    i = pl.program_id(0)
    dp = degp_ref[...]
    deg = _unpack_deg(dp[0] + dp[1], bs) + 1.0
    dsq = lax.rsqrt(deg)
    h = lax.dot_general(x_ref[...], w_ref[...],
                        dimension_numbers=(((1,), (1,)), ((), ())),
                        preferred_element_type=jnp.float32) + b_ref[...]
    rid = i * bs + lax.broadcasted_iota(jnp.int32, (bs, 1), 0)
    hd = jnp.where(rid < n, dsq * h, 0.0)
    half = ncls // 2
    hd_ref[0] = hd[:, :half]
    hd_ref[1] = hd[:, half:]


def _finish_body(bs, parts_ref, degp_ref, out_ref):
    dp = degp_ref[...]
    deg = _unpack_deg(dp[0] + dp[1], bs) + 1.0
    dsq = lax.rsqrt(deg)
    pre0 = parts_ref[0, 0] + parts_ref[0, 1]
    pre1 = parts_ref[1, 0] + parts_ref[1, 1]
    pre = dsq * jnp.concatenate([pre0, pre1], axis=1)
    m = jnp.max(pre, axis=1, keepdims=True)
    e = jnp.exp(pre - m)
    ssum = jnp.sum(e, axis=1, keepdims=True)
    out_ref[...] = pre - m - jnp.log(ssum)


def kernel(x, edge_index, W, b):
    n, nfeat = x.shape
    ncls = W.shape[0]
    e = edge_index.shape[1]
    half = ncls // 2

    rpt = -(-(n + 1) // NS)
    rpt = -(-rpt // 32) * 32
    n_pad = rpt * NS
    e_pad = -(-e // (4 * NW * CHUNK)) * (4 * NW * CHUNK)

    row = edge_index[0]
    col = edge_index[1]
    pad_e = e_pad - e
    rowp = jnp.concatenate(
        [row, jnp.full((pad_e,), n, jnp.int32)]).reshape(e_pad // CHUNK, CHUNK)
    colp = jnp.concatenate(
        [col, jnp.full((pad_e,), n, jnp.int32)]).reshape(e_pad // CHUNK, CHUNK)
    x_pad = jnp.pad(x, ((0, n_pad - n), (0, 0)))
    b2 = b.reshape(1, ncls)

    degp = _make_deg_kernel(e_pad, n_pad, rpt)(colp)

    bs = 1024
    bsp = bs // CHUNK
    grid = n_pad // bs
    hd = pl.pallas_call(
        functools.partial(_linear_body, n, bs, ncls),
        grid=(grid,),
        in_specs=[
            pl.BlockSpec((bs, nfeat), lambda i: (i, 0)),
            pl.BlockSpec((ncls, nfeat), lambda i: (0, 0)),
            pl.BlockSpec((1, ncls), lambda i: (0, 0)),
            pl.BlockSpec((NC, bsp, CHUNK), lambda i: (0, i, 0)),
        ],
        out_specs=pl.BlockSpec((2, bs, half), lambda i: (0, i, 0)),
        out_shape=jax.ShapeDtypeStruct((2, n_pad, half), jnp.float32),
    )(x_pad, W, b2, degp)

    parts = _make_agg_kernel(e_pad, n_pad, rpt, ncls)(rowp, colp, hd)

    out = pl.pallas_call(
        functools.partial(_finish_body, bs),
        grid=(grid,),
        in_specs=[
            pl.BlockSpec((2, NC, bs, half), lambda i: (0, 0, i, 0)),
            pl.BlockSpec((NC, bsp, CHUNK), lambda i: (0, i, 0)),
        ],
        out_specs=pl.BlockSpec((bs, ncls), lambda i: (i, 0)),
        out_shape=jax.ShapeDtypeStruct((n_pad, ncls), jnp.float32),
    )(parts, degp)

    return out[:n]


# speedup vs baseline: 12.0407x; 1.1839x over previous
import functools

import jax
import jax.numpy as jnp
from jax import lax
from jax.experimental import pallas as pl
from jax.experimental.pallas import tpu as pltpu
from jax.experimental.pallas import tpu_sc as plsc

NC = 2
NS = 16
NW = NC * NS
CHUNK = 128


def _mesh():
    return plsc.VectorSubcoreMesh(core_axis_name="c", subcore_axis_name="s",
                                  num_cores=NC, num_subcores=NS)


def _make_deg_kernel(e_pad, n_pad, rpt):
    nch = e_pad // (NW * CHUNK)
    nrows = n_pad // CHUNK
    rpc = nrows // NS

    @functools.partial(
        pl.kernel,
        out_type=jax.ShapeDtypeStruct((NC, nrows, CHUNK), jnp.float32),
        mesh=_mesh(),
        compiler_params=pltpu.CompilerParams(use_tc_tiling_on_sc=False,
                                             needs_layout_passes=False),
        scratch_types=[
            pltpu.VMEM((nch, CHUNK), jnp.int32),
            pltpu.VMEM((n_pad,), jnp.float32),
            pltpu.VMEM((nrows, CHUNK), jnp.float32),
            pltpu.VMEM((nrows,), jnp.int32),
            pltpu.VMEM_SHARED((nrows, CHUNK), jnp.float32),
        ],
    )
    def deg_kernel(col_hbm, out_hbm, colbuf2, degflat, degloc, ibuf, deg2):
        c = lax.axis_index("c")
        s = lax.axis_index("s")
        wid = s * NC + c

        def fill_zero(i, _):
            for j in range(CHUNK // 16):
                degflat[pl.ds(i * CHUNK + j * 16, 16)] = jnp.zeros(
                    (16,), jnp.float32)
                degloc[i, j * 16:(j + 1) * 16] = jnp.zeros((16,), jnp.float32)
            return 0

        lax.fori_loop(0, nrows, fill_zero, 0)
        for i in range(nrows // 16):
            ibuf[i * 16:(i + 1) * 16] = lax.iota(jnp.int32, 16) + i * 16
        pltpu.sync_copy(degloc.at[pl.ds(0, rpc)], deg2.at[pl.ds(s * rpc, rpc)])
        pltpu.sync_copy(col_hbm.at[pl.ds(wid * nch, nch)], colbuf2)
        plsc.subcore_barrier()

        ones16 = jnp.ones((16,), jnp.float32)

        def ebody(k, _):
            for j in range(CHUNK // 16):
                idx = colbuf2[k, j * 16:(j + 1) * 16]
                plsc.addupdate_scatter(degflat, [idx], ones16)
            return 0

        lax.fori_loop(0, nch, ebody, 0)

        def repack(i, _):
            for j in range(CHUNK // 16):
                degloc[i, j * 16:(j + 1) * 16] = degflat[
                    pl.ds(i * CHUNK + j * 16, 16)]
            return 0

        lax.fori_loop(0, nrows, repack, 0)
        pltpu.sync_copy(degloc, deg2.at[ibuf], add=True)
        plsc.subcore_barrier()
        pltpu.sync_copy(deg2.at[pl.ds(s * rpc, rpc)],
                        degloc.at[pl.ds(0, rpc)])
        pltpu.sync_copy(degloc.at[pl.ds(0, rpc)],
                        out_hbm.at[c, pl.ds(s * rpc, rpc)])

    return deg_kernel


def _make_agg_kernel(e_pad, n_pad, rpt, ncls):
    nch = e_pad // (NW * CHUNK)
    half = ncls // 2

    @functools.partial(
        pl.kernel,
        out_type=jax.ShapeDtypeStruct((NC, n_pad, CHUNK), jnp.float32),
        mesh=_mesh(),
        compiler_params=pltpu.CompilerParams(use_tc_tiling_on_sc=False),
        scratch_types=[
            pltpu.VMEM((nch, CHUNK), jnp.int32),
            pltpu.VMEM((nch, CHUNK), jnp.int32),
            pltpu.VMEM((CHUNK, half), jnp.float32),
            pltpu.VMEM((CHUNK, half), jnp.float32),
            pltpu.VMEM((CHUNK, half), jnp.float32),
            pltpu.VMEM((CHUNK, half), jnp.float32),
            pltpu.VMEM((rpt, half), jnp.float32),
            pltpu.VMEM_SHARED((n_pad, half), jnp.float32),
            pltpu.VMEM_SHARED((n_pad, half), jnp.float32),
            [pltpu.SemaphoreType.DMA] * 4,
            [pltpu.SemaphoreType.DMA] * 4,
        ],
    )
    def agg_kernel(row_hbm, col_hbm, hd_hbm, out_hbm,
                   colbuf2, rowbuf2, r0, r1, r2, r3, zbuf, agg, hds, gs, ss):
        c = lax.axis_index("c")
        s = lax.axis_index("s")
        wid = s * NC + c
        bufs = (r0, r1, r2, r3)

        pltpu.sync_copy(col_hbm.at[pl.ds(wid * nch, nch)], colbuf2)
        pltpu.sync_copy(row_hbm.at[pl.ds(wid * nch, nch)], rowbuf2)

        for p in range(2):
            def fill_zero(i, _):
                for j in range(half // 16):
                    zbuf[i, j * 16:(j + 1) * 16] = jnp.zeros((16,),
                                                             jnp.float32)
                return 0

            @pl.when(c == 1)
            def _():
                lax.fori_loop(0, rpt, fill_zero, 0)
                pltpu.sync_copy(zbuf, agg.at[pl.ds(s * rpt, rpt)])

            pltpu.sync_copy(
                hd_hbm.at[pl.ds(s * rpt, rpt), pl.ds(p * half, half)], zbuf)
            pltpu.sync_copy(zbuf, hds.at[pl.ds(s * rpt, rpt)])

            @pl.when(c == 0)
            def _():
                pltpu.sync_copy(zbuf, agg.at[pl.ds(s * rpt, rpt)])

            plsc.subcore_barrier()

            for b in range(4):
                pltpu.async_copy(hds.at[colbuf2.at[b]], bufs[b], gs[b])

            def ebody(k4, _):
                for b in range(4):
                    kk = k4 * 4 + b
                    rb = bufs[b]
                    pltpu.make_async_copy(
                        hd_hbm.at[pl.ds(0, CHUNK), pl.ds(0, half)],
                        rb, gs[b]).wait()
                    pltpu.async_copy(rb, agg.at[rowbuf2.at[kk]], ss[b],
                                     add=True)
                    nxt = kk + 4

                    @pl.when(nxt < nch)
                    def _():
                        pltpu.make_async_copy(
                            rb, agg.at[pl.ds(0, CHUNK)], ss[b]).wait()
                        pltpu.async_copy(hds.at[colbuf2.at[nxt]], rb, gs[b])
                return 0

            lax.fori_loop(0, nch // 4, ebody, 0)
            for b in range(4):
                pltpu.make_async_copy(
                    bufs[b], agg.at[pl.ds(0, CHUNK)], ss[b]).wait()
            plsc.subcore_barrier()
            pltpu.sync_copy(agg.at[pl.ds(s * rpt, rpt)], zbuf)
            pltpu.sync_copy(
                zbuf,
                out_hbm.at[c, pl.ds(s * rpt, rpt), pl.ds(p * half, half)])
            plsc.subcore_barrier()

    return agg_kernel


def _unpack_deg(dp, bs):
    bsp = dp.shape[0]
    rid = lax.broadcasted_iota(jnp.int32, (bs, 1), 0)
    rsel = (lax.broadcasted_iota(jnp.int32, (bs, bsp), 1)
            == (rid >> 7)).astype(jnp.float32)
    d = lax.dot_general(rsel, dp, dimension_numbers=(((1,), (0,)), ((), ())),
                        preferred_element_type=jnp.float32)
    lsel = (lax.broadcasted_iota(jnp.int32, (bs, CHUNK), 1) == (rid & 127))
    return jnp.sum(jnp.where(lsel, d, 0.0), axis=1, keepdims=True)


def _linear_body(n, bs, ncls, x_ref, w_ref, b_ref, degp_ref, hd_ref):
    i = pl.program_id(0)
    dp = degp_ref[...]
    deg = _unpack_deg(dp[0] + dp[1], bs) + 1.0
    dsq = lax.rsqrt(deg)
    h = lax.dot_general(x_ref[...], w_ref[...],
                        dimension_numbers=(((1,), (1,)), ((), ())),
                        preferred_element_type=jnp.float32) + b_ref[...]
    rid = i * bs + lax.broadcasted_iota(jnp.int32, (bs, 1), 0)
    hd_ref[...] = jnp.where(rid < n, dsq * h, 0.0)


def _finish_body(bs, ncls, parts_ref, degp_ref, out_ref):
    dp = degp_ref[...]
    deg = _unpack_deg(dp[0] + dp[1], bs) + 1.0
    dsq = lax.rsqrt(deg)
    pre = dsq * (parts_ref[0][:, :ncls] + parts_ref[1][:, :ncls])
    m = jnp.max(pre, axis=1, keepdims=True)
    e = jnp.exp(pre - m)
    ssum = jnp.sum(e, axis=1, keepdims=True)
    out_ref[...] = pre - m - jnp.log(ssum)


def kernel(x, edge_index, W, b):
    n, nfeat = x.shape
    ncls = W.shape[0]
    e = edge_index.shape[1]
    half = ncls // 2

    rpt = -(-(n + 1) // NS)
    rpt = -(-rpt // 32) * 32
    n_pad = rpt * NS
    e_pad = -(-e // (4 * NW * CHUNK)) * (4 * NW * CHUNK)

    row = edge_index[0]
    col = edge_index[1]
    pad_e = e_pad - e
    rowp = jnp.concatenate(
        [row, jnp.full((pad_e,), n, jnp.int32)]).reshape(e_pad // CHUNK, CHUNK)
    colp = jnp.concatenate(
        [col, jnp.full((pad_e,), n, jnp.int32)]).reshape(e_pad // CHUNK, CHUNK)
    x_pad = jnp.pad(x, ((0, n_pad - n), (0, 0)))
    w128 = jnp.pad(W, ((0, CHUNK - ncls), (0, 0)))
    b128 = jnp.pad(b, (0, CHUNK - ncls)).reshape(1, CHUNK)

    degp = _make_deg_kernel(e_pad, n_pad, rpt)(colp)

    bs = 1024
    bsp = bs // CHUNK
    grid = n_pad // bs
    hd = pl.pallas_call(
        functools.partial(_linear_body, n, bs, ncls),
        grid=(grid,),
        in_specs=[
            pl.BlockSpec((bs, nfeat), lambda i: (i, 0)),
            pl.BlockSpec((CHUNK, nfeat), lambda i: (0, 0)),
            pl.BlockSpec((1, CHUNK), lambda i: (0, 0)),
            pl.BlockSpec((NC, bsp, CHUNK), lambda i: (0, i, 0)),
        ],
        out_specs=pl.BlockSpec((bs, CHUNK), lambda i: (i, 0)),
        out_shape=jax.ShapeDtypeStruct((n_pad, CHUNK), jnp.float32),
    )(x_pad, w128, b128, degp)

    parts = _make_agg_kernel(e_pad, n_pad, rpt, ncls)(rowp, colp, hd)

    out = pl.pallas_call(
        functools.partial(_finish_body, bs, ncls),
        grid=(grid,),
        in_specs=[
            pl.BlockSpec((NC, bs, CHUNK), lambda i: (0, i, 0)),
            pl.BlockSpec((NC, bsp, CHUNK), lambda i: (0, i, 0)),
        ],
        out_specs=pl.BlockSpec((bs, ncls), lambda i: (i, 0)),
        out_shape=jax.ShapeDtypeStruct((n_pad, ncls), jnp.float32),
    )(parts, degp)

    return out[:n]


# speedup vs baseline: 13.2126x; 1.0973x over previous
import functools

import jax
import jax.numpy as jnp
from jax import lax
from jax.experimental import pallas as pl
from jax.experimental.pallas import tpu as pltpu
from jax.experimental.pallas import tpu_sc as plsc

NC = 2
NS = 16
NW = NC * NS
CHUNK = 128


def _mesh():
    return plsc.VectorSubcoreMesh(core_axis_name="c", subcore_axis_name="s",
                                  num_cores=NC, num_subcores=NS)


def _split(wid, tt, lo, a):
    hi = lo + 4
    start = jnp.where(wid < a, wid * hi, a * hi + (wid - a) * lo)
    ncht = jnp.where(wid < a, hi, lo)
    clamped = jnp.maximum(jnp.minimum(start, tt - hi), 0)
    return clamped, start - clamped, ncht


def _make_deg_kernel(tt, lo, a, n_pad, rpt):
    hi = lo + 4
    nrows = n_pad // CHUNK
    rpc = nrows // NS

    @functools.partial(
        pl.kernel,
        out_type=jax.ShapeDtypeStruct((NC, nrows, CHUNK), jnp.float32),
        mesh=_mesh(),
        compiler_params=pltpu.CompilerParams(use_tc_tiling_on_sc=False,
                                             needs_layout_passes=False),
        scratch_types=[
            pltpu.VMEM((hi, CHUNK), jnp.int32),
            pltpu.VMEM((n_pad,), jnp.float32),
            pltpu.VMEM((nrows, CHUNK), jnp.float32),
            pltpu.VMEM((nrows,), jnp.int32),
            pltpu.VMEM_SHARED((nrows, CHUNK), jnp.float32),
        ],
    )
    def deg_kernel(col_hbm, out_hbm, colbuf2, degflat, degloc, ibuf, deg2):
        c = lax.axis_index("c")
        s = lax.axis_index("s")
        wid = s * NC + c

        def fill_zero(i, _):
            for j in range(CHUNK // 16):
                degflat[pl.ds(i * CHUNK + j * 16, 16)] = jnp.zeros(
                    (16,), jnp.float32)
                degloc[i, j * 16:(j + 1) * 16] = jnp.zeros((16,), jnp.float32)
            return 0

        lax.fori_loop(0, nrows, fill_zero, 0)
        for i in range(nrows // 16):
            ibuf[i * 16:(i + 1) * 16] = lax.iota(jnp.int32, 16) + i * 16
        pltpu.sync_copy(degloc.at[pl.ds(0, rpc)], deg2.at[pl.ds(s * rpc, rpc)])
        base, off, ncht = _split(wid, tt, lo, a)
        pltpu.sync_copy(col_hbm.at[pl.ds(base, hi)], colbuf2)
        plsc.subcore_barrier()

        ones16 = jnp.ones((16,), jnp.float32)

        def ebody(k, _):
            for j in range(CHUNK // 16):
                idx = colbuf2[k, j * 16:(j + 1) * 16]
                plsc.addupdate_scatter(degflat, [idx], ones16)
            return 0

        lax.fori_loop(off, off + ncht, ebody, 0)

        def repack(i, _):
            for j in range(CHUNK // 16):
                degloc[i, j * 16:(j + 1) * 16] = degflat[
                    pl.ds(i * CHUNK + j * 16, 16)]
            return 0

        lax.fori_loop(0, nrows, repack, 0)
        pltpu.sync_copy(degloc, deg2.at[ibuf], add=True)
        plsc.subcore_barrier()
        pltpu.sync_copy(deg2.at[pl.ds(s * rpc, rpc)],
                        degloc.at[pl.ds(0, rpc)])
        pltpu.sync_copy(degloc.at[pl.ds(0, rpc)],
                        out_hbm.at[c, pl.ds(s * rpc, rpc)])

    return deg_kernel


def _make_agg_kernel(tt, lo, a, n_pad, rpt, ncls):
    hi = lo + 4
    half = ncls // 2

    @functools.partial(
        pl.kernel,
        out_type=jax.ShapeDtypeStruct((NC, n_pad, CHUNK), jnp.float32),
        mesh=_mesh(),
        compiler_params=pltpu.CompilerParams(use_tc_tiling_on_sc=False),
        scratch_types=[
            pltpu.VMEM((hi, CHUNK), jnp.int32),
            pltpu.VMEM((hi, CHUNK), jnp.int32),
            pltpu.VMEM((CHUNK, half), jnp.float32),
            pltpu.VMEM((CHUNK, half), jnp.float32),
            pltpu.VMEM((CHUNK, half), jnp.float32),
            pltpu.VMEM((CHUNK, half), jnp.float32),
            pltpu.VMEM((rpt, half), jnp.float32),
            pltpu.VMEM_SHARED((n_pad, half), jnp.float32),
            pltpu.VMEM_SHARED((n_pad, half), jnp.float32),
            [pltpu.SemaphoreType.DMA] * 4,
            [pltpu.SemaphoreType.DMA] * 4,
        ],
    )
    def agg_kernel(row_hbm, col_hbm, hd_hbm, out_hbm,
                   colbuf2, rowbuf2, r0, r1, r2, r3, zbuf, agg, hds,
                   gs, ss):
        c = lax.axis_index("c")
        s = lax.axis_index("s")
        wid = s * NC + c
        bufs = (r0, r1, r2, r3)

        base, off, ncht = _split(wid, tt, lo, a)
        pltpu.sync_copy(col_hbm.at[pl.ds(base, hi)], colbuf2)
        pltpu.sync_copy(row_hbm.at[pl.ds(base, hi)], rowbuf2)

        for p in range(2):
            def fill_zero(i, _):
                for j in range(half // 16):
                    zbuf[i, j * 16:(j + 1) * 16] = jnp.zeros((16,),
                                                             jnp.float32)
                return 0

            @pl.when(c == 1)
            def _():
                lax.fori_loop(0, rpt, fill_zero, 0)
                pltpu.sync_copy(zbuf, agg.at[pl.ds(s * rpt, rpt)])

            pltpu.sync_copy(
                hd_hbm.at[pl.ds(s * rpt, rpt), pl.ds(p * half, half)], zbuf)
            pltpu.sync_copy(zbuf, hds.at[pl.ds(s * rpt, rpt)])

            @pl.when(c == 0)
            def _():
                pltpu.sync_copy(zbuf, agg.at[pl.ds(s * rpt, rpt)])

            plsc.subcore_barrier()

            for b in range(4):
                pltpu.async_copy(hds.at[colbuf2.at[off + b]], bufs[b], gs[b])

            def ebody(k4, _):
                for b in range(4):
                    kk = off + k4 * 4 + b
                    rb = bufs[b]
                    pltpu.make_async_copy(
                        hd_hbm.at[pl.ds(0, CHUNK), pl.ds(0, half)],
                        rb, gs[b]).wait()
                    pltpu.async_copy(rb, agg.at[rowbuf2.at[kk]], ss[b],
                                     add=True)
                    nxt = kk + 4

                    @pl.when(nxt < off + ncht)
                    def _():
                        pltpu.make_async_copy(
                            rb, agg.at[pl.ds(0, CHUNK)], ss[b]).wait()
                        pltpu.async_copy(hds.at[colbuf2.at[nxt]], rb, gs[b])
                return 0

            lax.fori_loop(0, ncht // 4, ebody, 0)
            for b in range(4):
                pltpu.make_async_copy(
                    bufs[b], agg.at[pl.ds(0, CHUNK)], ss[b]).wait()
            plsc.subcore_barrier()
            pltpu.sync_copy(agg.at[pl.ds(s * rpt, rpt)], zbuf)
            pltpu.sync_copy(
                zbuf,
                out_hbm.at[c, pl.ds(s * rpt, rpt), pl.ds(p * half, half)])
            plsc.subcore_barrier()

    return agg_kernel


def _unpack_deg(dp, bs):
    bsp = dp.shape[0]
    rid = lax.broadcasted_iota(jnp.int32, (bs, 1), 0)
    rsel = (lax.broadcasted_iota(jnp.int32, (bs, bsp), 1)
            == (rid >> 7)).astype(jnp.float32)
    d = lax.dot_general(rsel, dp, dimension_numbers=(((1,), (0,)), ((), ())),
                        preferred_element_type=jnp.float32)
    lsel = (lax.broadcasted_iota(jnp.int32, (bs, CHUNK), 1) == (rid & 127))
    return jnp.sum(jnp.where(lsel, d, 0.0), axis=1, keepdims=True)


def _linear_body(n, bs, ncls, x_ref, w_ref, b_ref, degp_ref, hd_ref):
    i = pl.program_id(0)
    dp = degp_ref[...]
    deg = _unpack_deg(dp[0] + dp[1], bs) + 1.0
    dsq = lax.rsqrt(deg)
    h = lax.dot_general(x_ref[...], w_ref[...],
                        dimension_numbers=(((1,), (1,)), ((), ())),
                        preferred_element_type=jnp.float32) + b_ref[...]
    rid = i * bs + lax.broadcasted_iota(jnp.int32, (bs, 1), 0)
    hd_ref[...] = jnp.where(rid < n, dsq * h, 0.0)


def _finish_body(bs, ncls, parts_ref, degp_ref, out_ref):
    dp = degp_ref[...]
    deg = _unpack_deg(dp[0] + dp[1], bs) + 1.0
    dsq = lax.rsqrt(deg)
    pre = dsq * (parts_ref[0][:, :ncls] + parts_ref[1][:, :ncls])
    m = jnp.max(pre, axis=1, keepdims=True)
    e = jnp.exp(pre - m)
    ssum = jnp.sum(e, axis=1, keepdims=True)
    out_ref[...] = pre - m - jnp.log(ssum)


def kernel(x, edge_index, W, b):
    n, nfeat = x.shape
    ncls = W.shape[0]
    e = edge_index.shape[1]
    half = ncls // 2

    rpt = -(-(n + 1) // NS)
    rpt = -(-rpt // 32) * 32
    n_pad = rpt * NS

    row = edge_index[0]
    col = edge_index[1]
    pad_e = (-e) % (4 * CHUNK)
    if pad_e:
        row = jnp.concatenate([row, jnp.full((pad_e,), n, jnp.int32)])
        col = jnp.concatenate([col, jnp.full((pad_e,), n, jnp.int32)])
    tt = (e + pad_e) // CHUNK
    rowp = row.reshape(tt, CHUNK)
    colp = col.reshape(tt, CHUNK)
    lo = (tt // NW) // 4 * 4
    aa = (tt - NW * lo) // 4
    w128 = jnp.pad(W, ((0, CHUNK - ncls), (0, 0)))
    b128 = jnp.pad(b, (0, CHUNK - ncls)).reshape(1, CHUNK)

    degp = _make_deg_kernel(tt, lo, aa, n_pad, rpt)(colp)

    bs = 1024
    bsp = bs // CHUNK
    grid = n_pad // bs
    hd = pl.pallas_call(
        functools.partial(_linear_body, n, bs, ncls),
        grid=(grid,),
        in_specs=[
            pl.BlockSpec((bs, nfeat), lambda i: (i, 0)),
            pl.BlockSpec((CHUNK, nfeat), lambda i: (0, 0)),
            pl.BlockSpec((1, CHUNK), lambda i: (0, 0)),
            pl.BlockSpec((NC, bsp, CHUNK), lambda i: (0, i, 0)),
        ],
        out_specs=pl.BlockSpec((bs, CHUNK), lambda i: (i, 0)),
        out_shape=jax.ShapeDtypeStruct((n_pad, CHUNK), jnp.float32),
    )(x, w128, b128, degp)

    parts = _make_agg_kernel(tt, lo, aa, n_pad, rpt, ncls)(rowp, colp, hd)

    out = pl.pallas_call(
        functools.partial(_finish_body, bs, ncls),
        grid=(grid,),
        in_specs=[
            pl.BlockSpec((NC, bs, CHUNK), lambda i: (0, i, 0)),
            pl.BlockSpec((NC, bsp, CHUNK), lambda i: (0, i, 0)),
        ],
        out_specs=pl.BlockSpec((bs, ncls), lambda i: (i, 0)),
        out_shape=jax.ShapeDtypeStruct((n_pad, ncls), jnp.float32),
    )(parts, degp)

    return out[:n]


# speedup vs baseline: 14.4477x; 1.0935x over previous
import functools

import jax
import jax.numpy as jnp
from jax import lax
from jax.experimental import pallas as pl
from jax.experimental.pallas import tpu as pltpu
from jax.experimental.pallas import tpu_sc as plsc

NC = 2
NS = 16
NW = NC * NS
CHUNK = 128


def _mesh():
    return plsc.VectorSubcoreMesh(core_axis_name="c", subcore_axis_name="s",
                                  num_cores=NC, num_subcores=NS)


def _plan(tt, f0):
    t0 = min(int(round(tt * f0 / 4)) * 4, tt)
    g0, r0 = divmod(t0 // 4, NS)
    g1, r1 = divmod((tt - t0) // 4, NS)
    hi = 4 * max(g0 + (1 if r0 else 0), g1 + (1 if r1 else 0))
    return g0, r0, t0, g1, r1, hi


def _split(c, s, plan, tt):
    g0, r0, t0, g1, r1, hi = plan
    s0 = 4 * (g0 * s + jnp.minimum(s, r0))
    s1 = t0 + 4 * (g1 * s + jnp.minimum(s, r1))
    start = jnp.where(c == 0, s0, s1)
    ncht = 4 * jnp.where(c == 0, g0 + (s < r0), g1 + (s < r1))
    clamped = jnp.maximum(jnp.minimum(start, tt - hi), 0)
    return clamped, start - clamped, ncht


def _make_deg_kernel(tt, plan, n_pad, rpt):
    hi = plan[5]
    nrows = n_pad // CHUNK
    rpc = nrows // NS

    @functools.partial(
        pl.kernel,
        out_type=jax.ShapeDtypeStruct((NC, nrows, CHUNK), jnp.float32),
        mesh=_mesh(),
        compiler_params=pltpu.CompilerParams(use_tc_tiling_on_sc=False,
                                             needs_layout_passes=False),
        scratch_types=[
            pltpu.VMEM((hi, CHUNK), jnp.int32),
            pltpu.VMEM((n_pad,), jnp.float32),
            pltpu.VMEM((nrows, CHUNK), jnp.float32),
            pltpu.VMEM((nrows,), jnp.int32),
            pltpu.VMEM_SHARED((nrows, CHUNK), jnp.float32),
        ],
    )
    def deg_kernel(edge_hbm, out_hbm, colbuf2, degflat, degloc, ibuf, deg2):
        c = lax.axis_index("c")
        s = lax.axis_index("s")

        def fill_zero(i, _):
            for j in range(CHUNK // 16):
                degflat[pl.ds(i * CHUNK + j * 16, 16)] = jnp.zeros(
                    (16,), jnp.float32)
                degloc[i, j * 16:(j + 1) * 16] = jnp.zeros((16,), jnp.float32)
            return 0

        lax.fori_loop(0, nrows, fill_zero, 0)
        for i in range(nrows // 16):
            ibuf[i * 16:(i + 1) * 16] = lax.iota(jnp.int32, 16) + i * 16
        pltpu.sync_copy(degloc.at[pl.ds(0, rpc)], deg2.at[pl.ds(s * rpc, rpc)])
        base, off, ncht = _split(c, s, plan, tt)
        pltpu.sync_copy(edge_hbm.at[1, pl.ds(base, hi)], colbuf2)
        plsc.subcore_barrier()

        ones16 = jnp.ones((16,), jnp.float32)

        def ebody(k, _):
            for j in range(CHUNK // 16):
                idx = colbuf2[k, j * 16:(j + 1) * 16]
                plsc.addupdate_scatter(degflat, [idx], ones16)
            return 0

        lax.fori_loop(off, off + ncht, ebody, 0)

        def repack(i, _):
            for j in range(CHUNK // 16):
                degloc[i, j * 16:(j + 1) * 16] = degflat[
                    pl.ds(i * CHUNK + j * 16, 16)]
            return 0

        lax.fori_loop(0, nrows, repack, 0)
        pltpu.sync_copy(degloc, deg2.at[ibuf], add=True)
        plsc.subcore_barrier()
        pltpu.sync_copy(deg2.at[pl.ds(s * rpc, rpc)],
                        degloc.at[pl.ds(0, rpc)])
        pltpu.sync_copy(degloc.at[pl.ds(0, rpc)],
                        out_hbm.at[c, pl.ds(s * rpc, rpc)])

    return deg_kernel


def _make_agg_kernel(tt, plan, n_pad, rpt, ncls):
    hi = plan[5]
    half = ncls // 2

    @functools.partial(
        pl.kernel,
        out_type=jax.ShapeDtypeStruct((NC, n_pad, CHUNK), jnp.float32),
        mesh=_mesh(),
        compiler_params=pltpu.CompilerParams(use_tc_tiling_on_sc=False),
        scratch_types=[
            pltpu.VMEM((hi, CHUNK), jnp.int32),
            pltpu.VMEM((hi, CHUNK), jnp.int32),
            pltpu.VMEM((CHUNK, half), jnp.float32),
            pltpu.VMEM((CHUNK, half), jnp.float32),
            pltpu.VMEM((CHUNK, half), jnp.float32),
            pltpu.VMEM((CHUNK, half), jnp.float32),
            pltpu.VMEM((rpt, half), jnp.float32),
            pltpu.VMEM_SHARED((n_pad, half), jnp.float32),
            pltpu.VMEM_SHARED((n_pad, half), jnp.float32),
            [pltpu.SemaphoreType.DMA] * 4,
            [pltpu.SemaphoreType.DMA] * 4,
        ],
    )
    def agg_kernel(edge_hbm, hd_hbm, out_hbm,
                   colbuf2, rowbuf2, r0, r1, r2, r3, zbuf, agg, hds,
                   gs, ss):
        c = lax.axis_index("c")
        s = lax.axis_index("s")
        bufs = (r0, r1, r2, r3)

        base, off, ncht = _split(c, s, plan, tt)
        pltpu.sync_copy(edge_hbm.at[1, pl.ds(base, hi)], colbuf2)
        pltpu.sync_copy(edge_hbm.at[0, pl.ds(base, hi)], rowbuf2)

        for p in range(2):
            def fill_zero(i, _):
                for j in range(half // 16):
                    zbuf[i, j * 16:(j + 1) * 16] = jnp.zeros((16,),
                                                             jnp.float32)
                return 0

            @pl.when(c == 1)
            def _():
                lax.fori_loop(0, rpt, fill_zero, 0)
                pltpu.sync_copy(zbuf, agg.at[pl.ds(s * rpt, rpt)])

            pltpu.sync_copy(
                hd_hbm.at[pl.ds(s * rpt, rpt), pl.ds(p * half, half)], zbuf)
            pltpu.sync_copy(zbuf, hds.at[pl.ds(s * rpt, rpt)])

            @pl.when(c == 0)
            def _():
                pltpu.sync_copy(zbuf, agg.at[pl.ds(s * rpt, rpt)])

            plsc.subcore_barrier()

            for b in range(4):
                pltpu.async_copy(hds.at[colbuf2.at[off + b]], bufs[b], gs[b])

            def ebody(k4, _):
                for b in range(4):
                    kk = off + k4 * 4 + b
                    rb = bufs[b]
                    pltpu.make_async_copy(
                        hd_hbm.at[pl.ds(0, CHUNK), pl.ds(0, half)],
                        rb, gs[b]).wait()
                    pltpu.async_copy(rb, agg.at[rowbuf2.at[kk]], ss[b],
                                     add=True)
                    nxt = kk + 4

                    @pl.when(nxt < off + ncht)
                    def _():
                        pltpu.make_async_copy(
                            rb, agg.at[pl.ds(0, CHUNK)], ss[b]).wait()
                        pltpu.async_copy(hds.at[colbuf2.at[nxt]], rb, gs[b])
                return 0

            lax.fori_loop(0, ncht // 4, ebody, 0)
            for b in range(4):
                pltpu.make_async_copy(
                    bufs[b], agg.at[pl.ds(0, CHUNK)], ss[b]).wait()
            plsc.subcore_barrier()
            pltpu.sync_copy(agg.at[pl.ds(s * rpt, rpt)], zbuf)
            pltpu.sync_copy(
                zbuf,
                out_hbm.at[c, pl.ds(s * rpt, rpt), pl.ds(p * half, half)])
            plsc.subcore_barrier()

    return agg_kernel


def _unpack_deg(dp, bs):
    bsp = dp.shape[0]
    rid = lax.broadcasted_iota(jnp.int32, (bs, 1), 0)
    rsel = (lax.broadcasted_iota(jnp.int32, (bs, bsp), 1)
            == (rid >> 7)).astype(jnp.float32)
    d = lax.dot_general(rsel, dp, dimension_numbers=(((1,), (0,)), ((), ())),
                        preferred_element_type=jnp.float32)
    lsel = (lax.broadcasted_iota(jnp.int32, (bs, CHUNK), 1) == (rid & 127))
    return jnp.sum(jnp.where(lsel, d, 0.0), axis=1, keepdims=True)


def _linear_body(n, bs, ncls, x_ref, w_ref, b_ref, degp_ref, hd_ref):
    i = pl.program_id(0)
    dp = degp_ref[...]
    deg = _unpack_deg(dp[0] + dp[1], bs) + 1.0
    dsq = lax.rsqrt(deg)
    h = lax.dot_general(x_ref[...], w_ref[...],
                        dimension_numbers=(((1,), (1,)), ((), ())),
                        preferred_element_type=jnp.float32) + b_ref[...]
    rid = i * bs + lax.broadcasted_iota(jnp.int32, (bs, 1), 0)
    hd_ref[...] = jnp.where(rid < n, dsq * h, 0.0)


def _finish_body(bs, ncls, parts_ref, degp_ref, out_ref):
    dp = degp_ref[...]
    deg = _unpack_deg(dp[0] + dp[1], bs) + 1.0
    dsq = lax.rsqrt(deg)
    pre = dsq * (parts_ref[0][:, :ncls] + parts_ref[1][:, :ncls])
    m = jnp.max(pre, axis=1, keepdims=True)
    e = jnp.exp(pre - m)
    ssum = jnp.sum(e, axis=1, keepdims=True)
    out_ref[...] = pre - m - jnp.log(ssum)


def kernel(x, edge_index, W, b):
    n, nfeat = x.shape
    ncls = W.shape[0]
    e = edge_index.shape[1]
    half = ncls // 2

    rpt = -(-(n + 1) // NS)
    rpt = -(-rpt // 32) * 32
    n_pad = rpt * NS

    pad_e = (-e) % (4 * CHUNK)
    ei = edge_index
    if pad_e:
        ei = jnp.concatenate(
            [ei, jnp.full((2, pad_e), n, jnp.int32)], axis=1)
    tt = (e + pad_e) // CHUNK
    edges = ei.reshape(2, tt, CHUNK)
    plan = _plan(tt, 0.52)
    w128 = jnp.pad(W, ((0, CHUNK - ncls), (0, 0)))
    b128 = jnp.pad(b, (0, CHUNK - ncls)).reshape(1, CHUNK)

    degp = _make_deg_kernel(tt, plan, n_pad, rpt)(edges)

    bs = 1024
    bsp = bs // CHUNK
    grid = n_pad // bs
    hd = pl.pallas_call(
        functools.partial(_linear_body, n, bs, ncls),
        grid=(grid,),
        in_specs=[
            pl.BlockSpec((bs, nfeat), lambda i: (i, 0)),
            pl.BlockSpec((CHUNK, nfeat), lambda i: (0, 0)),
            pl.BlockSpec((1, CHUNK), lambda i: (0, 0)),
            pl.BlockSpec((NC, bsp, CHUNK), lambda i: (0, i, 0)),
        ],
        out_specs=pl.BlockSpec((bs, CHUNK), lambda i: (i, 0)),
        out_shape=jax.ShapeDtypeStruct((n_pad, CHUNK), jnp.float32),
    )(x, w128, b128, degp)

    parts = _make_agg_kernel(tt, plan, n_pad, rpt, ncls)(edges, hd)

    out = pl.pallas_call(
        functools.partial(_finish_body, bs, ncls),
        grid=(grid,),
        in_specs=[
            pl.BlockSpec((NC, bs, CHUNK), lambda i: (0, i, 0)),
            pl.BlockSpec((NC, bsp, CHUNK), lambda i: (0, i, 0)),
        ],
        out_specs=pl.BlockSpec((bs, ncls), lambda i: (i, 0)),
        out_shape=jax.ShapeDtypeStruct((n, ncls), jnp.float32),
    )(parts, degp)

    return out
